# Initial kernel scaffold; baseline (speedup 1.0000x reference)
import jax
import jax.numpy as jnp
from jax.experimental import pallas as pl


def kernel(x, edge_index, batch, t_fc_W, t_fc_b, ln_g, ln_b, qW, kW, vW, g_fc_W, g_fc_b, conv_W, conv_b, fc_W, fc_b):
    raise NotImplementedError("write your pallas kernel here")


# speedup vs baseline: 12.6676x; 12.6676x over previous
import functools

import jax
import jax.numpy as jnp
from jax import lax
from jax.experimental import pallas as pl
from jax.experimental.pallas import tpu as pltpu
from jax.experimental.pallas import tpu_sc as plsc

_BN = float(1.0 / (1.0 + 1e-5) ** 0.5)
_RB = 1000
_CH = 128
_NP = 10240
_ZR = 128


def _ln(h, g, b):
    m = jnp.mean(h, axis=-1, keepdims=True)
    v = jnp.mean((h - m) ** 2, axis=-1, keepdims=True)
    return (h - m) * lax.rsqrt(v + 1e-5) * g + b


def _qknorm(y):
    y = jnp.where(y == 0.0, 1e-6, y)
    return y * lax.rsqrt(jnp.sum(y * y, axis=-1, keepdims=True))


def _k1_body(x, tW, tb, g0, b0, gW, gb, cW0, dA, dB, xt0, xg0, hp0):
    xb = x[...]
    h = jnp.dot(xb, tW[...], preferred_element_type=jnp.float32) + tb[...]
    xt0[...] = jnp.maximum(_ln(h, g0[...], b0[...]), 0.0)
    g = jnp.maximum(_BN * (jnp.dot(xb, gW[...], preferred_element_type=jnp.float32) + gb[...]), 0.0)
    xg0[...] = g
    dinv = lax.rsqrt(dA[:, :1] + dB[:, :1] + 1.0)
    hp0[...] = dinv * jnp.dot(g, cW0[...], preferred_element_type=jnp.float32)


def _attn_stats_body(xt, kW, vW, kvs, ksum):
    i = pl.program_id(0)
    xb = xt[...]
    ks = _qknorm(jnp.dot(xb, kW[...], preferred_element_type=jnp.float32))
    vs = jnp.dot(xb, vW[...], preferred_element_type=jnp.float32)
    pkvs = lax.dot_general(ks, vs, (((0,), (0,)), ((), ())),
                           preferred_element_type=jnp.float32)
    pksum = jnp.sum(ks, axis=0, keepdims=True)

    @pl.when(i == 0)
    def _():
        kvs[...] = pkvs
        ksum[...] = pksum

    @pl.when(i != 0)
    def _():
        kvs[...] = kvs[...] + pkvs
        ksum[...] = ksum[...] + pksum


def _attn_apply_body(nn, xt, qW, vW, kvs, ksum, g, b, out):
    xb = xt[...]
    qs = _qknorm(jnp.dot(xb, qW[...], preferred_element_type=jnp.float32))
    vs = jnp.dot(xb, vW[...], preferred_element_type=jnp.float32)
    num = jnp.dot(qs, kvs[...], preferred_element_type=jnp.float32) + nn * vs
    den = jnp.sum(qs * ksum[...], axis=-1, keepdims=True) + nn
    a = (num / den + xb) * 0.5
    out[...] = jnp.maximum(_ln(a, g[...], b[...]), 0.0)


def _gcn_mid_body(aggA, aggB, hp, dA, dB, cb, xg0, cWn, hp_out):
    dinv = lax.rsqrt(dA[:, :1] + dB[:, :1] + 1.0)
    s = dinv * (aggA[...] + aggB[...] + hp[...]) + cb[...]
    xg = jnp.maximum(_BN * s, 0.0) + xg0[...]
    hp_out[...] = dinv * jnp.dot(xg, cWn[...], preferred_element_type=jnp.float32)


def _gcn_final_body(aggA, aggB, hp, dA, dB, cb, xg0, xtf, fW, fb, out):
    dinv = lax.rsqrt(dA[:, :1] + dB[:, :1] + 1.0)
    s = dinv * (aggA[...] + aggB[...] + hp[...]) + cb[...]
    xg = jnp.maximum(_BN * s, 0.0) + xg0[...]
    comb = 0.5 * xg + 0.5 * xtf[...]
    out[...] = jnp.dot(comb, fW[...], preferred_element_type=jnp.float32) + fb[...]


def _row_spec(h):
    return pl.BlockSpec((_RB, h), lambda i: (i, 0))


def _full_spec(r, c):
    return pl.BlockSpec((r, c), lambda i: (0, 0))


def _sc_deg(dst, ones, zeros):
    e = dst.shape[0]
    per_w = e // 32
    nfull, tail = divmod(per_w, _CH)

    @functools.partial(
        pl.kernel,
        out_type=jax.ShapeDtypeStruct((2, _NP, 128), jnp.float32),
        mesh=plsc.VectorSubcoreMesh(core_axis_name="c", subcore_axis_name="s"),
        scratch_types=[
            pltpu.VMEM_SHARED((_NP, 128), jnp.float32),
            pltpu.VMEM((_CH,), jnp.int32),
            pltpu.VMEM((tail,), jnp.int32),
            pltpu.VMEM((_CH, 128), jnp.float32),
            pltpu.VMEM((_ZR, 128), jnp.float32),
        ],
    )
    def k(dst_hbm, ones_hbm, zeros_hbm, out_hbm, acc, idxv, idxt, onesv, zv):
        c = lax.axis_index("c")
        s = lax.axis_index("s")
        pltpu.sync_copy(zeros_hbm, zv)
        pltpu.sync_copy(ones_hbm, onesv)
        for j in range(5):
            pltpu.sync_copy(zv, acc.at[pl.ds(s * 640 + j * _ZR, _ZR)])
        plsc.subcore_barrier()
        base0 = (c * 16 + s) * per_w

        def body(i, carry):
            pltpu.sync_copy(dst_hbm.at[pl.ds(base0 + i * _CH, _CH)], idxv)
            pltpu.sync_copy(onesv, acc.at[idxv], add=True)
            return carry

        lax.fori_loop(0, nfull, body, 0)
        if tail:
            pltpu.sync_copy(dst_hbm.at[pl.ds(base0 + nfull * _CH, tail)], idxt)
            pltpu.sync_copy(onesv.at[pl.ds(0, tail)], acc.at[idxt], add=True)
        plsc.subcore_barrier()
        pltpu.sync_copy(acc.at[pl.ds(s * 640, 640)], out_hbm.at[c, pl.ds(s * 640, 640)])

    return k(dst, ones, zeros)


def _sc_agg(h, src, dst, zeros):
    n, hd = h.shape
    e = src.shape[0]
    per_w = e // 32
    nfull, tail = divmod(per_w, _CH)

    @functools.partial(
        pl.kernel,
        out_type=jax.ShapeDtypeStruct((2, _NP, hd), jnp.float32),
        mesh=plsc.VectorSubcoreMesh(core_axis_name="c", subcore_axis_name="s"),
        scratch_types=[
            pltpu.VMEM_SHARED((_NP, hd), jnp.float32),
            pltpu.VMEM((_CH,), jnp.int32),
            pltpu.VMEM((_CH,), jnp.int32),
            pltpu.VMEM((_CH, hd), jnp.float32),
            pltpu.VMEM((tail,), jnp.int32),
            pltpu.VMEM((tail,), jnp.int32),
            pltpu.VMEM((tail, hd), jnp.float32),
            pltpu.VMEM((_ZR, hd), jnp.float32),
            pltpu.SemaphoreType.DMA,
        ],
    )
    def k(h_hbm, src_hbm, dst_hbm, zeros_hbm, out_hbm,
          acc, srcv, dstv, rowsv, srct, dstt, rowst, zv, sem):
        c = lax.axis_index("c")
        s = lax.axis_index("s")
        pltpu.sync_copy(zeros_hbm, zv)
        for j in range(5):
            pltpu.sync_copy(zv, acc.at[pl.ds(s * 640 + j * _ZR, _ZR)])
        plsc.subcore_barrier()
        base0 = (c * 16 + s) * per_w

        def body(i, carry):
            base = base0 + i * _CH
            pltpu.sync_copy(src_hbm.at[pl.ds(base, _CH)], srcv)
            pltpu.sync_copy(dst_hbm.at[pl.ds(base, _CH)], dstv)
            pltpu.async_copy(h_hbm.at[srcv], rowsv, sem).wait()
            pltpu.sync_copy(rowsv, acc.at[dstv], add=True)
            return carry

        lax.fori_loop(0, nfull, body, 0)
        if tail:
            base = base0 + nfull * _CH
            pltpu.sync_copy(src_hbm.at[pl.ds(base, tail)], srct)
            pltpu.sync_copy(dst_hbm.at[pl.ds(base, tail)], dstt)
            pltpu.async_copy(h_hbm.at[srct], rowst, sem).wait()
            pltpu.sync_copy(rowst, acc.at[dstt], add=True)
        plsc.subcore_barrier()
        pltpu.sync_copy(acc.at[pl.ds(s * 640, 640)], out_hbm.at[c, pl.ds(s * 640, 640)])

    return k(h, src, dst, zeros)


def kernel(x, edge_index, batch, t_fc_W, t_fc_b, ln_g, ln_b, qW, kW, vW,
           g_fc_W, g_fc_b, conv_W, conv_b, fc_W, fc_b):
    n, d = x.shape
    h = t_fc_W.shape[1]
    out_d = fc_W.shape[1]
    tl = qW.shape[0]
    gl = conv_W.shape[0]
    grid = (n // _RB,)
    del batch

    src = edge_index[0]
    dst = edge_index[1]
    ones = jnp.ones((_CH, h), jnp.float32)
    zeros = jnp.zeros((_ZR, h), jnp.float32)

    degp = _sc_deg(dst, ones, zeros)
    degA, degB = degp[0], degp[1]

    rsp = _row_spec(h)
    wsp = _full_spec(h, h)
    bsp = _full_spec(1, h)
    dsp = pl.BlockSpec((_RB, 128), lambda i: (i, 0))
    fsd = jax.ShapeDtypeStruct((n, h), jnp.float32)

    xt, xg0, hp = pl.pallas_call(
        _k1_body,
        grid=grid,
        in_specs=[rsp if d == h else _row_spec(d),
                  _full_spec(d, h), bsp, bsp, bsp,
                  _full_spec(d, h), bsp, wsp, dsp, dsp],
        out_specs=[rsp, rsp, rsp],
        out_shape=[fsd, fsd, fsd],
    )(x, t_fc_W, t_fc_b.reshape(1, h), ln_g[0].reshape(1, h), ln_b[0].reshape(1, h),
      g_fc_W, g_fc_b.reshape(1, h), conv_W[0], degA, degB)

    for i in range(tl):
        kvs, ksum = pl.pallas_call(
            _attn_stats_body,
            grid=grid,
            in_specs=[rsp, wsp, wsp],
            out_specs=[_full_spec(h, h), bsp],
            out_shape=[jax.ShapeDtypeStruct((h, h), jnp.float32),
                       jax.ShapeDtypeStruct((1, h), jnp.float32)],
        )(xt, kW[i], vW[i])
        xt = pl.pallas_call(
            functools.partial(_attn_apply_body, float(n)),
            grid=grid,
            in_specs=[rsp, wsp, wsp, _full_spec(h, h), bsp, bsp, bsp],
            out_specs=rsp,
            out_shape=fsd,
        )(xt, qW[i], vW[i], kvs, ksum, ln_g[i + 1].reshape(1, h), ln_b[i + 1].reshape(1, h))

    for i in range(gl):
        agg = _sc_agg(hp, src, dst, zeros)
        if i + 1 < gl:
            hp = pl.pallas_call(
                _gcn_mid_body,
                grid=grid,
                in_specs=[rsp, rsp, rsp, dsp, dsp, bsp, rsp, wsp],
                out_specs=rsp,
                out_shape=fsd,
            )(agg[0], agg[1], hp, degA, degB, conv_b[i].reshape(1, h), xg0, conv_W[i + 1])
        else:
            out = pl.pallas_call(
                _gcn_final_body,
                grid=grid,
                in_specs=[rsp, rsp, rsp, dsp, dsp, bsp, rsp, rsp,
                          _full_spec(h, out_d), _full_spec(1, out_d)],
                out_specs=_row_spec(out_d),
                out_shape=jax.ShapeDtypeStruct((n, out_d), jnp.float32),
            )(agg[0], agg[1], hp, degA, degB, conv_b[i].reshape(1, h), xg0, xt,
              fc_W, fc_b.reshape(1, out_d))
    return out


# speedup vs baseline: 14.9152x; 1.1774x over previous
import functools

import jax
import jax.numpy as jnp
from jax import lax
from jax.experimental import pallas as pl
from jax.experimental.pallas import tpu as pltpu
from jax.experimental.pallas import tpu_sc as plsc

_BN = float(1.0 / (1.0 + 1e-5) ** 0.5)
_RB = 1000
_CH = 128
_NP = 10240
_ZR = 64


def _ln(h, g, b):
    m = jnp.mean(h, axis=-1, keepdims=True)
    v = jnp.mean((h - m) ** 2, axis=-1, keepdims=True)
    return (h - m) * lax.rsqrt(v + 1e-5) * g + b


def _qknorm(y):
    y = jnp.where(y == 0.0, 1e-6, y)
    return y * lax.rsqrt(jnp.sum(y * y, axis=-1, keepdims=True))


def _k1_body(x, tW, tb, g0, b0, gW, gb, cW0, dA, dB, xt0, xg0, hp0):
    xb = x[...]
    h = jnp.dot(xb, tW[...], preferred_element_type=jnp.float32) + tb[...]
    xt0[...] = jnp.maximum(_ln(h, g0[...], b0[...]), 0.0)
    g = jnp.maximum(_BN * (jnp.dot(xb, gW[...], preferred_element_type=jnp.float32) + gb[...]), 0.0)
    xg0[...] = g
    dinv = lax.rsqrt(dA[:, :1] + dB[:, :1] + 1.0)
    hp0[...] = dinv * jnp.dot(g, cW0[...], preferred_element_type=jnp.float32)


def _attn_stats_body(xt, kW, vW, kvs, ksum):
    i = pl.program_id(0)
    xb = xt[...]
    ks = _qknorm(jnp.dot(xb, kW[...], preferred_element_type=jnp.float32))
    vs = jnp.dot(xb, vW[...], preferred_element_type=jnp.float32)
    pkvs = lax.dot_general(ks, vs, (((0,), (0,)), ((), ())),
                           preferred_element_type=jnp.float32)
    pksum = jnp.sum(ks, axis=0, keepdims=True)

    @pl.when(i == 0)
    def _():
        kvs[...] = pkvs
        ksum[...] = pksum

    @pl.when(i != 0)
    def _():
        kvs[...] = kvs[...] + pkvs
        ksum[...] = ksum[...] + pksum


def _attn_apply_body(nn, xt, qW, vW, kvs, ksum, g, b, out):
    xb = xt[...]
    qs = _qknorm(jnp.dot(xb, qW[...], preferred_element_type=jnp.float32))
    vs = jnp.dot(xb, vW[...], preferred_element_type=jnp.float32)
    num = jnp.dot(qs, kvs[...], preferred_element_type=jnp.float32) + nn * vs
    den = jnp.sum(qs * ksum[...], axis=-1, keepdims=True) + nn
    a = (num / den + xb) * 0.5
    out[...] = jnp.maximum(_ln(a, g[...], b[...]), 0.0)


def _gcn_mid_body(aggA, aggB, hp, dA, dB, cb, xg0, cWn, hp_out):
    dinv = lax.rsqrt(dA[:, :1] + dB[:, :1] + 1.0)
    s = dinv * (aggA[...] + aggB[...] + hp[...]) + cb[...]
    xg = jnp.maximum(_BN * s, 0.0) + xg0[...]
    hp_out[...] = dinv * jnp.dot(xg, cWn[...], preferred_element_type=jnp.float32)


def _gcn_final_body(aggA, aggB, hp, dA, dB, cb, xg0, xtf, fW, fb, out):
    dinv = lax.rsqrt(dA[:, :1] + dB[:, :1] + 1.0)
    s = dinv * (aggA[...] + aggB[...] + hp[...]) + cb[...]
    xg = jnp.maximum(_BN * s, 0.0) + xg0[...]
    comb = 0.5 * xg + 0.5 * xtf[...]
    out[...] = jnp.dot(comb, fW[...], preferred_element_type=jnp.float32) + fb[...]


def _row_spec(h):
    return pl.BlockSpec((_RB, h), lambda i: (i, 0))


def _full_spec(r, c):
    return pl.BlockSpec((r, c), lambda i: (0, 0))


def _sc_deg(dst, ones, zeros):
    e = dst.shape[0]
    per_w = e // 32
    nfull, tail = divmod(per_w, _CH)

    @functools.partial(
        pl.kernel,
        out_type=jax.ShapeDtypeStruct((2, _NP, 128), jnp.float32),
        mesh=plsc.VectorSubcoreMesh(core_axis_name="c", subcore_axis_name="s"),
        scratch_types=[
            pltpu.VMEM_SHARED((_NP, 128), jnp.float32),
            pltpu.VMEM((_CH,), jnp.int32),
            pltpu.VMEM((tail,), jnp.int32),
            pltpu.VMEM((_CH, 128), jnp.float32),
            pltpu.VMEM((_ZR, 128), jnp.float32),
        ],
    )
    def k(dst_hbm, ones_hbm, zeros_hbm, out_hbm, acc, idxv, idxt, onesv, zv):
        c = lax.axis_index("c")
        s = lax.axis_index("s")
        pltpu.sync_copy(zeros_hbm, zv)
        pltpu.sync_copy(ones_hbm, onesv)
        for j in range(10):
            pltpu.sync_copy(zv, acc.at[pl.ds(s * 640 + j * _ZR, _ZR)])
        plsc.subcore_barrier()
        base0 = (c * 16 + s) * per_w

        def body(i, carry):
            pltpu.sync_copy(dst_hbm.at[pl.ds(base0 + i * _CH, _CH)], idxv)
            pltpu.sync_copy(onesv, acc.at[idxv], add=True)
            return carry

        lax.fori_loop(0, nfull, body, 0)
        if tail:
            pltpu.sync_copy(dst_hbm.at[pl.ds(base0 + nfull * _CH, tail)], idxt)
            pltpu.sync_copy(onesv.at[pl.ds(0, tail)], acc.at[idxt], add=True)
        plsc.subcore_barrier()
        pltpu.sync_copy(acc.at[pl.ds(s * 640, 640)], out_hbm.at[c, pl.ds(s * 640, 640)])

    return k(dst, ones, zeros)


def _sc_agg(h, src, dst, zeros):
    n, hd = h.shape
    e = src.shape[0]
    per_w = e // 32
    nfull, tail = divmod(per_w, _CH)

    npairs, odd = divmod(nfull, 2)

    @functools.partial(
        pl.kernel,
        out_type=jax.ShapeDtypeStruct((2, _NP, hd), jnp.float32),
        mesh=plsc.VectorSubcoreMesh(core_axis_name="c", subcore_axis_name="s"),
        scratch_types=[
            pltpu.VMEM_SHARED((_NP, hd), jnp.float32),
            pltpu.VMEM((_CH,), jnp.int32),
            pltpu.VMEM((_CH,), jnp.int32),
            pltpu.VMEM((_CH,), jnp.int32),
            pltpu.VMEM((_CH,), jnp.int32),
            pltpu.VMEM((_CH, hd), jnp.float32),
            pltpu.VMEM((_CH, hd), jnp.float32),
            pltpu.VMEM((tail,), jnp.int32),
            pltpu.VMEM((tail,), jnp.int32),
            pltpu.VMEM((tail, hd), jnp.float32),
            pltpu.VMEM((_ZR, hd), jnp.float32),
            pltpu.SemaphoreType.DMA,
            pltpu.SemaphoreType.DMA,
            pltpu.SemaphoreType.DMA,
            pltpu.SemaphoreType.DMA,
        ],
    )
    def k(h_hbm, src_hbm, dst_hbm, zeros_hbm, out_hbm,
          acc, src0, src1, dst0, dst1, rows0, rows1, srct, dstt, rowst, zv,
          g0, g1, s0, s1):
        c = lax.axis_index("c")
        s = lax.axis_index("s")
        srcv, dstv, rowsv = (src0, src1), (dst0, dst1), (rows0, rows1)
        gsem, ssem = (g0, g1), (s0, s1)
        pltpu.sync_copy(zeros_hbm, zv)
        for j in range(10):
            pltpu.sync_copy(zv, acc.at[pl.ds(s * 640 + j * _ZR, _ZR)])
        plsc.subcore_barrier()
        base0 = (c * 16 + s) * per_w

        def pair(i, carry):
            for b in range(2):
                base = base0 + (2 * i + b) * _CH

                @pl.when(i > 0)
                def _():
                    pltpu.make_async_copy(rowsv[b], acc.at[dstv[b]], ssem[b]).wait()

                pltpu.sync_copy(src_hbm.at[pl.ds(base, _CH)], srcv[b])
                pltpu.sync_copy(dst_hbm.at[pl.ds(base, _CH)], dstv[b])
                pltpu.async_copy(h_hbm.at[srcv[b]], rowsv[b], gsem[b]).wait()
                pltpu.async_copy(rowsv[b], acc.at[dstv[b]], ssem[b], add=True)
            return carry

        lax.fori_loop(0, npairs, pair, 0)
        for b in range(2):
            pltpu.make_async_copy(rowsv[b], acc.at[dstv[b]], ssem[b]).wait()
        if odd:
            base = base0 + (nfull - 1) * _CH
            pltpu.sync_copy(src_hbm.at[pl.ds(base, _CH)], src0)
            pltpu.sync_copy(dst_hbm.at[pl.ds(base, _CH)], dst0)
            pltpu.async_copy(h_hbm.at[src0], rows0, g0).wait()
            pltpu.sync_copy(rows0, acc.at[dst0], add=True)
        if tail:
            base = base0 + nfull * _CH
            pltpu.sync_copy(src_hbm.at[pl.ds(base, tail)], srct)
            pltpu.sync_copy(dst_hbm.at[pl.ds(base, tail)], dstt)
            pltpu.async_copy(h_hbm.at[srct], rowst, g0).wait()
            pltpu.sync_copy(rowst, acc.at[dstt], add=True)
        plsc.subcore_barrier()
        pltpu.sync_copy(acc.at[pl.ds(s * 640, 640)], out_hbm.at[c, pl.ds(s * 640, 640)])

    return k(h, src, dst, zeros)


def kernel(x, edge_index, batch, t_fc_W, t_fc_b, ln_g, ln_b, qW, kW, vW,
           g_fc_W, g_fc_b, conv_W, conv_b, fc_W, fc_b):
    n, d = x.shape
    h = t_fc_W.shape[1]
    out_d = fc_W.shape[1]
    tl = qW.shape[0]
    gl = conv_W.shape[0]
    grid = (n // _RB,)
    del batch

    src = edge_index[0]
    dst = edge_index[1]
    ones = jnp.ones((_CH, h), jnp.float32)
    zeros = jnp.zeros((_ZR, h), jnp.float32)

    degp = _sc_deg(dst, ones, zeros)
    degA, degB = degp[0], degp[1]

    rsp = _row_spec(h)
    wsp = _full_spec(h, h)
    bsp = _full_spec(1, h)
    dsp = pl.BlockSpec((_RB, 128), lambda i: (i, 0))
    fsd = jax.ShapeDtypeStruct((n, h), jnp.float32)

    xt, xg0, hp = pl.pallas_call(
        _k1_body,
        grid=grid,
        in_specs=[rsp if d == h else _row_spec(d),
                  _full_spec(d, h), bsp, bsp, bsp,
                  _full_spec(d, h), bsp, wsp, dsp, dsp],
        out_specs=[rsp, rsp, rsp],
        out_shape=[fsd, fsd, fsd],
    )(x, t_fc_W, t_fc_b.reshape(1, h), ln_g[0].reshape(1, h), ln_b[0].reshape(1, h),
      g_fc_W, g_fc_b.reshape(1, h), conv_W[0], degA, degB)

    for i in range(tl):
        kvs, ksum = pl.pallas_call(
            _attn_stats_body,
            grid=grid,
            in_specs=[rsp, wsp, wsp],
            out_specs=[_full_spec(h, h), bsp],
            out_shape=[jax.ShapeDtypeStruct((h, h), jnp.float32),
                       jax.ShapeDtypeStruct((1, h), jnp.float32)],
        )(xt, kW[i], vW[i])
        xt = pl.pallas_call(
            functools.partial(_attn_apply_body, float(n)),
            grid=grid,
            in_specs=[rsp, wsp, wsp, _full_spec(h, h), bsp, bsp, bsp],
            out_specs=rsp,
            out_shape=fsd,
        )(xt, qW[i], vW[i], kvs, ksum, ln_g[i + 1].reshape(1, h), ln_b[i + 1].reshape(1, h))

    for i in range(gl):
        agg = _sc_agg(hp, src, dst, zeros)
        if i + 1 < gl:
            hp = pl.pallas_call(
                _gcn_mid_body,
                grid=grid,
                in_specs=[rsp, rsp, rsp, dsp, dsp, bsp, rsp, wsp],
                out_specs=rsp,
                out_shape=fsd,
            )(agg[0], agg[1], hp, degA, degB, conv_b[i].reshape(1, h), xg0, conv_W[i + 1])
        else:
            out = pl.pallas_call(
                _gcn_final_body,
                grid=grid,
                in_specs=[rsp, rsp, rsp, dsp, dsp, bsp, rsp, rsp,
                          _full_spec(h, out_d), _full_spec(1, out_d)],
                out_specs=_row_spec(out_d),
                out_shape=jax.ShapeDtypeStruct((n, out_d), jnp.float32),
            )(agg[0], agg[1], hp, degA, degB, conv_b[i].reshape(1, h), xg0, xt,
              fc_W, fc_b.reshape(1, out_d))
    return out


# speedup vs baseline: 19.7266x; 1.3226x over previous
import functools

import jax
import jax.numpy as jnp
from jax import lax
from jax.experimental import pallas as pl
from jax.experimental.pallas import tpu as pltpu
from jax.experimental.pallas import tpu_sc as plsc

_BN = float(1.0 / (1.0 + 1e-5) ** 0.5)
_RB = 1000
_CH = 128
_NP = 10240
_ZR = 64


def _ln(h, g, b):
    m = jnp.mean(h, axis=-1, keepdims=True)
    v = jnp.mean((h - m) ** 2, axis=-1, keepdims=True)
    return (h - m) * lax.rsqrt(v + 1e-5) * g + b


def _qknorm(y):
    y = jnp.where(y == 0.0, 1e-6, y)
    return y * lax.rsqrt(jnp.sum(y * y, axis=-1, keepdims=True))


def _k1_body(x, tW, tb, g0, b0, gW, gb, cW0, dA, dB, xt0, xg0, hp0):
    xb = x[...]
    h = jnp.dot(xb, tW[...], preferred_element_type=jnp.float32) + tb[...]
    xt0[...] = jnp.maximum(_ln(h, g0[...], b0[...]), 0.0)
    g = jnp.maximum(_BN * (jnp.dot(xb, gW[...], preferred_element_type=jnp.float32) + gb[...]), 0.0)
    xg0[...] = g
    dinv = lax.rsqrt(dA[:, :1] + dB[:, :1] + 1.0)
    hp0[...] = dinv * jnp.dot(g, cW0[...], preferred_element_type=jnp.float32)


def _attn_stats_body(xt, kW, vW, kvs, ksum):
    i = pl.program_id(0)
    xb = xt[...]
    ks = _qknorm(jnp.dot(xb, kW[...], preferred_element_type=jnp.float32))
    vs = jnp.dot(xb, vW[...], preferred_element_type=jnp.float32)
    pkvs = lax.dot_general(ks, vs, (((0,), (0,)), ((), ())),
                           preferred_element_type=jnp.float32)
    pksum = jnp.sum(ks, axis=0, keepdims=True)

    @pl.when(i == 0)
    def _():
        kvs[...] = pkvs
        ksum[...] = pksum

    @pl.when(i != 0)
    def _():
        kvs[...] = kvs[...] + pkvs
        ksum[...] = ksum[...] + pksum


def _attn_apply_body(nn, xt, qW, vW, kvs, ksum, g, b, out):
    xb = xt[...]
    qs = _qknorm(jnp.dot(xb, qW[...], preferred_element_type=jnp.float32))
    vs = jnp.dot(xb, vW[...], preferred_element_type=jnp.float32)
    num = jnp.dot(qs, kvs[...], preferred_element_type=jnp.float32) + nn * vs
    den = jnp.sum(qs * ksum[...], axis=-1, keepdims=True) + nn
    a = (num / den + xb) * 0.5
    out[...] = jnp.maximum(_ln(a, g[...], b[...]), 0.0)


def _gcn_mid_body(aggA, aggB, hp, dA, dB, cb, xg0, cWn, hp_out):
    dinv = lax.rsqrt(dA[:, :1] + dB[:, :1] + 1.0)
    s = dinv * (aggA[...] + aggB[...] + hp[...]) + cb[...]
    xg = jnp.maximum(_BN * s, 0.0) + xg0[...]
    hp_out[...] = dinv * jnp.dot(xg, cWn[...], preferred_element_type=jnp.float32)


def _gcn_final_body(aggA, aggB, hp, dA, dB, cb, xg0, xtf, fW, fb, out):
    dinv = lax.rsqrt(dA[:, :1] + dB[:, :1] + 1.0)
    s = dinv * (aggA[...] + aggB[...] + hp[...]) + cb[...]
    xg = jnp.maximum(_BN * s, 0.0) + xg0[...]
    comb = 0.5 * xg + 0.5 * xtf[...]
    out[...] = jnp.dot(comb, fW[...], preferred_element_type=jnp.float32) + fb[...]


def _row_spec(h):
    return pl.BlockSpec((_RB, h), lambda i: (i, 0))


def _full_spec(r, c):
    return pl.BlockSpec((r, c), lambda i: (0, 0))


def _sc_deg(dst, ones, zeros):
    e = dst.shape[0]
    per_w = e // 32
    nfull, tail = divmod(per_w, _CH)

    @functools.partial(
        pl.kernel,
        out_type=jax.ShapeDtypeStruct((2, _NP, 128), jnp.float32),
        mesh=plsc.VectorSubcoreMesh(core_axis_name="c", subcore_axis_name="s"),
        scratch_types=[
            pltpu.VMEM_SHARED((_NP, 128), jnp.float32),
            pltpu.VMEM((_CH,), jnp.int32),
            pltpu.VMEM((tail,), jnp.int32),
            pltpu.VMEM((_CH, 128), jnp.float32),
            pltpu.VMEM((_ZR, 128), jnp.float32),
        ],
    )
    def k(dst_hbm, ones_hbm, zeros_hbm, out_hbm, acc, idxv, idxt, onesv, zv):
        c = lax.axis_index("c")
        s = lax.axis_index("s")
        pltpu.sync_copy(zeros_hbm, zv)
        pltpu.sync_copy(ones_hbm, onesv)
        for j in range(10):
            pltpu.sync_copy(zv, acc.at[pl.ds(s * 640 + j * _ZR, _ZR)])
        plsc.subcore_barrier()
        base0 = (c * 16 + s) * per_w

        def body(i, carry):
            pltpu.sync_copy(dst_hbm.at[pl.ds(base0 + i * _CH, _CH)], idxv)
            pltpu.sync_copy(onesv, acc.at[idxv], add=True)
            return carry

        lax.fori_loop(0, nfull, body, 0)
        if tail:
            pltpu.sync_copy(dst_hbm.at[pl.ds(base0 + nfull * _CH, tail)], idxt)
            pltpu.sync_copy(onesv.at[pl.ds(0, tail)], acc.at[idxt], add=True)
        plsc.subcore_barrier()
        pltpu.sync_copy(acc.at[pl.ds(s * 640, 640)], out_hbm.at[c, pl.ds(s * 640, 640)])

    return k(dst, ones, zeros)


def _sc_agg(h, src, dst, zeros):
    n, hd = h.shape
    e = src.shape[0]
    per_w = e // 32
    nfull, tail = divmod(per_w, _CH)

    assert nfull % 6 == 0, nfull

    @functools.partial(
        pl.kernel,
        out_type=jax.ShapeDtypeStruct((2, _NP, hd), jnp.float32),
        mesh=plsc.VectorSubcoreMesh(core_axis_name="c", subcore_axis_name="s"),
        scratch_types=[
            pltpu.VMEM_SHARED((_NP, hd), jnp.float32),
            pltpu.VMEM((3, _CH), jnp.int32),
            pltpu.VMEM((3, _CH), jnp.int32),
            pltpu.VMEM((_CH, hd), jnp.float32),
            pltpu.VMEM((_CH, hd), jnp.float32),
            pltpu.VMEM((tail,), jnp.int32),
            pltpu.VMEM((tail,), jnp.int32),
            pltpu.VMEM((tail, hd), jnp.float32),
            pltpu.VMEM((_ZR, hd), jnp.float32),
            [pltpu.SemaphoreType.DMA] * 3,
            [pltpu.SemaphoreType.DMA] * 3,
            [pltpu.SemaphoreType.DMA] * 2,
            [pltpu.SemaphoreType.DMA] * 2,
        ],
    )
    def k(h_hbm, src_hbm, dst_hbm, zeros_hbm, out_hbm,
          acc, srci, dsti, rows0, rows1, srct, dstt, rowst, zv,
          isems, jsems, gsem, ssem):
        c = lax.axis_index("c")
        s = lax.axis_index("s")
        rowsv = (rows0, rows1)
        pltpu.sync_copy(zeros_hbm, zv)
        for j in range(10):
            pltpu.sync_copy(zv, acc.at[pl.ds(s * 640 + j * _ZR, _ZR)])
        plsc.subcore_barrier()
        base0 = (c * 16 + s) * per_w

        def issue_idx(j, ib):
            base = base0 + j * _CH
            pltpu.async_copy(src_hbm.at[pl.ds(base, _CH)], srci.at[ib], isems[ib])
            pltpu.async_copy(dst_hbm.at[pl.ds(base, _CH)], dsti.at[ib], jsems[ib])

        def wait_idx(j, ib):
            base = base0 + j * _CH
            pltpu.make_async_copy(src_hbm.at[pl.ds(base, _CH)], srci.at[ib], isems[ib]).wait()
            pltpu.make_async_copy(dst_hbm.at[pl.ds(base, _CH)], dsti.at[ib], jsems[ib]).wait()

        issue_idx(0, 0)

        def six(i, carry):
            for u in range(6):
                j = 6 * i + u
                b = u % 2
                ib = u % 3

                @pl.when(j >= 2)
                def _():
                    pltpu.make_async_copy(rowsv[b], acc.at[dsti.at[(u + 1) % 3]], ssem[b]).wait()

                @pl.when(j + 1 < nfull)
                def _():
                    issue_idx(j + 1, (u + 1) % 3)

                wait_idx(j, ib)
                pltpu.async_copy(h_hbm.at[srci.at[ib]], rowsv[b], gsem[b]).wait()
                pltpu.async_copy(rowsv[b], acc.at[dsti.at[ib]], ssem[b], add=True)
            return carry

        lax.fori_loop(0, nfull // 6, six, 0)
        for b in range(2):
            pltpu.make_async_copy(rowsv[b], acc.at[dsti.at[b]], ssem[b]).wait()
        if tail:
            base = base0 + nfull * _CH
            pltpu.sync_copy(src_hbm.at[pl.ds(base, tail)], srct)
            pltpu.sync_copy(dst_hbm.at[pl.ds(base, tail)], dstt)
            pltpu.async_copy(h_hbm.at[srct], rowst, gsem[0]).wait()
            pltpu.sync_copy(rowst, acc.at[dstt], add=True)
        plsc.subcore_barrier()
        pltpu.sync_copy(acc.at[pl.ds(s * 640, 640)], out_hbm.at[c, pl.ds(s * 640, 640)])

    return k(h, src, dst, zeros)


def kernel(x, edge_index, batch, t_fc_W, t_fc_b, ln_g, ln_b, qW, kW, vW,
           g_fc_W, g_fc_b, conv_W, conv_b, fc_W, fc_b):
    n, d = x.shape
    h = t_fc_W.shape[1]
    out_d = fc_W.shape[1]
    tl = qW.shape[0]
    gl = conv_W.shape[0]
    grid = (n // _RB,)
    del batch

    src = edge_index[0]
    dst = edge_index[1]
    ones = jnp.ones((_CH, h), jnp.float32)
    zeros = jnp.zeros((_ZR, h), jnp.float32)

    degp = _sc_deg(dst, ones, zeros)
    degA, degB = degp[0], degp[1]

    rsp = _row_spec(h)
    wsp = _full_spec(h, h)
    bsp = _full_spec(1, h)
    dsp = pl.BlockSpec((_RB, 128), lambda i: (i, 0))
    fsd = jax.ShapeDtypeStruct((n, h), jnp.float32)

    xt, xg0, hp = pl.pallas_call(
        _k1_body,
        grid=grid,
        in_specs=[rsp if d == h else _row_spec(d),
                  _full_spec(d, h), bsp, bsp, bsp,
                  _full_spec(d, h), bsp, wsp, dsp, dsp],
        out_specs=[rsp, rsp, rsp],
        out_shape=[fsd, fsd, fsd],
    )(x, t_fc_W, t_fc_b.reshape(1, h), ln_g[0].reshape(1, h), ln_b[0].reshape(1, h),
      g_fc_W, g_fc_b.reshape(1, h), conv_W[0], degA, degB)

    for i in range(tl):
        kvs, ksum = pl.pallas_call(
            _attn_stats_body,
            grid=grid,
            in_specs=[rsp, wsp, wsp],
            out_specs=[_full_spec(h, h), bsp],
            out_shape=[jax.ShapeDtypeStruct((h, h), jnp.float32),
                       jax.ShapeDtypeStruct((1, h), jnp.float32)],
        )(xt, kW[i], vW[i])
        xt = pl.pallas_call(
            functools.partial(_attn_apply_body, float(n)),
            grid=grid,
            in_specs=[rsp, wsp, wsp, _full_spec(h, h), bsp, bsp, bsp],
            out_specs=rsp,
            out_shape=fsd,
        )(xt, qW[i], vW[i], kvs, ksum, ln_g[i + 1].reshape(1, h), ln_b[i + 1].reshape(1, h))

    for i in range(gl):
        agg = _sc_agg(hp, src, dst, zeros)
        if i + 1 < gl:
            hp = pl.pallas_call(
                _gcn_mid_body,
                grid=grid,
                in_specs=[rsp, rsp, rsp, dsp, dsp, bsp, rsp, wsp],
                out_specs=rsp,
                out_shape=fsd,
            )(agg[0], agg[1], hp, degA, degB, conv_b[i].reshape(1, h), xg0, conv_W[i + 1])
        else:
            out = pl.pallas_call(
                _gcn_final_body,
                grid=grid,
                in_specs=[rsp, rsp, rsp, dsp, dsp, bsp, rsp, rsp,
                          _full_spec(h, out_d), _full_spec(1, out_d)],
                out_specs=_row_spec(out_d),
                out_shape=jax.ShapeDtypeStruct((n, out_d), jnp.float32),
            )(agg[0], agg[1], hp, degA, degB, conv_b[i].reshape(1, h), xg0, xt,
              fc_W, fc_b.reshape(1, out_d))
    return out


# speedup vs baseline: 22.8060x; 1.1561x over previous
import functools

import jax
import jax.numpy as jnp
from jax import lax
from jax.experimental import pallas as pl
from jax.experimental.pallas import tpu as pltpu
from jax.experimental.pallas import tpu_sc as plsc

_BN = float(1.0 / (1.0 + 1e-5) ** 0.5)
_RB = 1000
_CH = 128
_NP = 10240
_ZR = 64


def _ln(h, g, b):
    m = jnp.mean(h, axis=-1, keepdims=True)
    v = jnp.mean((h - m) ** 2, axis=-1, keepdims=True)
    return (h - m) * lax.rsqrt(v + 1e-5) * g + b


def _qknorm(y):
    y = jnp.where(y == 0.0, 1e-6, y)
    return y * lax.rsqrt(jnp.sum(y * y, axis=-1, keepdims=True))


def _k1_body(x, tW, tb, g0, b0, gW, gb, cW0, dA, dB, xt0, xg0, hp0):
    xb = x[...]
    h = jnp.dot(xb, tW[...], preferred_element_type=jnp.float32) + tb[...]
    xt0[...] = jnp.maximum(_ln(h, g0[...], b0[...]), 0.0)
    g = jnp.maximum(_BN * (jnp.dot(xb, gW[...], preferred_element_type=jnp.float32) + gb[...]), 0.0)
    xg0[...] = g
    dinv = lax.rsqrt(dA[...] + dB[...] + 1.0)
    hp0[...] = dinv * jnp.dot(g, cW0[...], preferred_element_type=jnp.float32)


def _attn_stats_body(xt, kW, vW, kvs, ksum):
    i = pl.program_id(0)
    xb = xt[...]
    ks = _qknorm(jnp.dot(xb, kW[...], preferred_element_type=jnp.float32))
    vs = jnp.dot(xb, vW[...], preferred_element_type=jnp.float32)
    pkvs = lax.dot_general(ks, vs, (((0,), (0,)), ((), ())),
                           preferred_element_type=jnp.float32)
    pksum = jnp.sum(ks, axis=0, keepdims=True)

    @pl.when(i == 0)
    def _():
        kvs[...] = pkvs
        ksum[...] = pksum

    @pl.when(i != 0)
    def _():
        kvs[...] = kvs[...] + pkvs
        ksum[...] = ksum[...] + pksum


def _attn_apply_body(nn, xt, qW, vW, kvs, ksum, g, b, out):
    xb = xt[...]
    qs = _qknorm(jnp.dot(xb, qW[...], preferred_element_type=jnp.float32))
    vs = jnp.dot(xb, vW[...], preferred_element_type=jnp.float32)
    num = jnp.dot(qs, kvs[...], preferred_element_type=jnp.float32) + nn * vs
    den = jnp.sum(qs * ksum[...], axis=-1, keepdims=True) + nn
    a = (num / den + xb) * 0.5
    out[...] = jnp.maximum(_ln(a, g[...], b[...]), 0.0)


def _gcn_mid_body(aggA, aggB, hp, dA, dB, cb, xg0, cWn, hp_out):
    dinv = lax.rsqrt(dA[...] + dB[...] + 1.0)
    s = dinv * (aggA[...] + aggB[...] + hp[...]) + cb[...]
    xg = jnp.maximum(_BN * s, 0.0) + xg0[...]
    hp_out[...] = dinv * jnp.dot(xg, cWn[...], preferred_element_type=jnp.float32)


def _gcn_final_body(aggA, aggB, hp, dA, dB, cb, xg0, xtf, fW, fb, out):
    dinv = lax.rsqrt(dA[...] + dB[...] + 1.0)
    s = dinv * (aggA[...] + aggB[...] + hp[...]) + cb[...]
    xg = jnp.maximum(_BN * s, 0.0) + xg0[...]
    comb = 0.5 * xg + 0.5 * xtf[...]
    out[...] = jnp.dot(comb, fW[...], preferred_element_type=jnp.float32) + fb[...]


def _row_spec(h):
    return pl.BlockSpec((_RB, h), lambda i: (i, 0))


def _full_spec(r, c):
    return pl.BlockSpec((r, c), lambda i: (0, 0))


def _sc_deg(dst):
    e = dst.shape[0]
    per_w = e // 32
    nfull, tail = divmod(per_w, _CH)

    @functools.partial(
        pl.kernel,
        out_type=jax.ShapeDtypeStruct((2, _NP), jnp.float32),
        mesh=plsc.VectorSubcoreMesh(core_axis_name="c", subcore_axis_name="s"),
        scratch_types=[
            pltpu.VMEM_SHARED((_NP,), jnp.float32),
            pltpu.VMEM((3, _CH), jnp.int32),
            pltpu.VMEM((tail,), jnp.int32),
            pltpu.VMEM((_CH,), jnp.float32),
            pltpu.VMEM((640,), jnp.float32),
            [pltpu.SemaphoreType.DMA] * 3,
            [pltpu.SemaphoreType.DMA] * 2,
        ],
    )
    def k(dst_hbm, out_hbm, acc, idxv, idxt, onesv, zv, isems, ssem):
        c = lax.axis_index("c")
        s = lax.axis_index("s")
        ones16 = jnp.ones((16,), jnp.float32)
        zero16 = jnp.zeros((16,), jnp.float32)
        for r in range(_CH // 16):
            onesv[pl.ds(r * 16, 16)] = ones16
        for r in range(640 // 16):
            zv[pl.ds(r * 16, 16)] = zero16
        pltpu.sync_copy(zv, acc.at[pl.ds(s * 640, 640)])
        plsc.subcore_barrier()
        base0 = (c * 16 + s) * per_w

        def issue_idx(j, ib):
            pltpu.async_copy(dst_hbm.at[pl.ds(base0 + j * _CH, _CH)], idxv.at[ib], isems[ib])

        def wait_idx(j, ib):
            pltpu.make_async_copy(dst_hbm.at[pl.ds(base0 + j * _CH, _CH)], idxv.at[ib], isems[ib]).wait()

        issue_idx(0, 0)

        def six(i, carry):
            for u in range(6):
                j = 6 * i + u
                ib = u % 3
                b = u % 2

                @pl.when(j >= 2)
                def _():
                    pltpu.make_async_copy(onesv, acc.at[idxv.at[(u + 1) % 3]], ssem[b]).wait()

                @pl.when(j + 1 < nfull)
                def _():
                    issue_idx(j + 1, (u + 1) % 3)

                wait_idx(j, ib)
                pltpu.async_copy(onesv, acc.at[idxv.at[ib]], ssem[b], add=True)
            return carry

        lax.fori_loop(0, nfull // 6, six, 0)
        for b in range(2):
            pltpu.make_async_copy(onesv, acc.at[idxv.at[b]], ssem[b]).wait()
        if tail:
            pltpu.sync_copy(dst_hbm.at[pl.ds(base0 + nfull * _CH, tail)], idxt)
            pltpu.sync_copy(onesv.at[pl.ds(0, tail)], acc.at[idxt], add=True)
        plsc.subcore_barrier()
        pltpu.sync_copy(acc.at[pl.ds(s * 640, 640)], out_hbm.at[c, pl.ds(s * 640, 640)])

    return k(dst)


def _sc_agg(h, src, dst, zeros):
    n, hd = h.shape
    e = src.shape[0]
    per_w = e // 32
    nfull, tail = divmod(per_w, _CH)

    assert nfull % 6 == 0, nfull

    @functools.partial(
        pl.kernel,
        out_type=jax.ShapeDtypeStruct((2, _NP, hd), jnp.float32),
        mesh=plsc.VectorSubcoreMesh(core_axis_name="c", subcore_axis_name="s"),
        scratch_types=[
            pltpu.VMEM_SHARED((_NP, hd), jnp.float32),
            pltpu.VMEM((3, _CH), jnp.int32),
            pltpu.VMEM((3, _CH), jnp.int32),
            pltpu.VMEM((_CH, hd), jnp.float32),
            pltpu.VMEM((_CH, hd), jnp.float32),
            pltpu.VMEM((tail,), jnp.int32),
            pltpu.VMEM((tail,), jnp.int32),
            pltpu.VMEM((tail, hd), jnp.float32),
            pltpu.VMEM((_ZR, hd), jnp.float32),
            [pltpu.SemaphoreType.DMA] * 3,
            [pltpu.SemaphoreType.DMA] * 3,
            [pltpu.SemaphoreType.DMA] * 2,
            [pltpu.SemaphoreType.DMA] * 2,
        ],
    )
    def k(h_hbm, src_hbm, dst_hbm, zeros_hbm, out_hbm,
          acc, srci, dsti, rows0, rows1, srct, dstt, rowst, zv,
          isems, jsems, gsem, ssem):
        c = lax.axis_index("c")
        s = lax.axis_index("s")
        rowsv = (rows0, rows1)
        pltpu.sync_copy(zeros_hbm, zv)
        for j in range(10):
            pltpu.sync_copy(zv, acc.at[pl.ds(s * 640 + j * _ZR, _ZR)])
        plsc.subcore_barrier()
        base0 = (c * 16 + s) * per_w

        def issue_idx(j, ib):
            base = base0 + j * _CH
            pltpu.async_copy(src_hbm.at[pl.ds(base, _CH)], srci.at[ib], isems[ib])
            pltpu.async_copy(dst_hbm.at[pl.ds(base, _CH)], dsti.at[ib], jsems[ib])

        def wait_idx(j, ib):
            base = base0 + j * _CH
            pltpu.make_async_copy(src_hbm.at[pl.ds(base, _CH)], srci.at[ib], isems[ib]).wait()
            pltpu.make_async_copy(dst_hbm.at[pl.ds(base, _CH)], dsti.at[ib], jsems[ib]).wait()

        issue_idx(0, 0)

        def six(i, carry):
            for u in range(6):
                j = 6 * i + u
                b = u % 2
                ib = u % 3

                @pl.when(j >= 2)
                def _():
                    pltpu.make_async_copy(rowsv[b], acc.at[dsti.at[(u + 1) % 3]], ssem[b]).wait()

                @pl.when(j + 1 < nfull)
                def _():
                    issue_idx(j + 1, (u + 1) % 3)

                wait_idx(j, ib)
                pltpu.async_copy(h_hbm.at[srci.at[ib]], rowsv[b], gsem[b]).wait()
                pltpu.async_copy(rowsv[b], acc.at[dsti.at[ib]], ssem[b], add=True)
            return carry

        lax.fori_loop(0, nfull // 6, six, 0)
        for b in range(2):
            pltpu.make_async_copy(rowsv[b], acc.at[dsti.at[b]], ssem[b]).wait()
        if tail:
            base = base0 + nfull * _CH
            pltpu.sync_copy(src_hbm.at[pl.ds(base, tail)], srct)
            pltpu.sync_copy(dst_hbm.at[pl.ds(base, tail)], dstt)
            pltpu.async_copy(h_hbm.at[srct], rowst, gsem[0]).wait()
            pltpu.sync_copy(rowst, acc.at[dstt], add=True)
        plsc.subcore_barrier()
        pltpu.sync_copy(acc.at[pl.ds(s * 640, 640)], out_hbm.at[c, pl.ds(s * 640, 640)])

    return k(h, src, dst, zeros)


def kernel(x, edge_index, batch, t_fc_W, t_fc_b, ln_g, ln_b, qW, kW, vW,
           g_fc_W, g_fc_b, conv_W, conv_b, fc_W, fc_b):
    n, d = x.shape
    h = t_fc_W.shape[1]
    out_d = fc_W.shape[1]
    tl = qW.shape[0]
    gl = conv_W.shape[0]
    grid = (n // _RB,)
    del batch

    src = edge_index[0]
    dst = edge_index[1]
    zeros = jnp.zeros((_ZR, h), jnp.float32)

    degp = _sc_deg(dst)
    degA = degp[0].reshape(_NP, 1)
    degB = degp[1].reshape(_NP, 1)

    rsp = _row_spec(h)
    wsp = _full_spec(h, h)
    bsp = _full_spec(1, h)
    dsp = pl.BlockSpec((_RB, 1), lambda i: (i, 0))
    fsd = jax.ShapeDtypeStruct((n, h), jnp.float32)

    xt, xg0, hp = pl.pallas_call(
        _k1_body,
        grid=grid,
        in_specs=[rsp if d == h else _row_spec(d),
                  _full_spec(d, h), bsp, bsp, bsp,
                  _full_spec(d, h), bsp, wsp, dsp, dsp],
        out_specs=[rsp, rsp, rsp],
        out_shape=[fsd, fsd, fsd],
    )(x, t_fc_W, t_fc_b.reshape(1, h), ln_g[0].reshape(1, h), ln_b[0].reshape(1, h),
      g_fc_W, g_fc_b.reshape(1, h), conv_W[0], degA, degB)

    for i in range(tl):
        kvs, ksum = pl.pallas_call(
            _attn_stats_body,
            grid=grid,
            in_specs=[rsp, wsp, wsp],
            out_specs=[_full_spec(h, h), bsp],
            out_shape=[jax.ShapeDtypeStruct((h, h), jnp.float32),
                       jax.ShapeDtypeStruct((1, h), jnp.float32)],
        )(xt, kW[i], vW[i])
        xt = pl.pallas_call(
            functools.partial(_attn_apply_body, float(n)),
            grid=grid,
            in_specs=[rsp, wsp, wsp, _full_spec(h, h), bsp, bsp, bsp],
            out_specs=rsp,
            out_shape=fsd,
        )(xt, qW[i], vW[i], kvs, ksum, ln_g[i + 1].reshape(1, h), ln_b[i + 1].reshape(1, h))

    for i in range(gl):
        agg = _sc_agg(hp, src, dst, zeros)
        if i + 1 < gl:
            hp = pl.pallas_call(
                _gcn_mid_body,
                grid=grid,
                in_specs=[rsp, rsp, rsp, dsp, dsp, bsp, rsp, wsp],
                out_specs=rsp,
                out_shape=fsd,
            )(agg[0], agg[1], hp, degA, degB, conv_b[i].reshape(1, h), xg0, conv_W[i + 1])
        else:
            out = pl.pallas_call(
                _gcn_final_body,
                grid=grid,
                in_specs=[rsp, rsp, rsp, dsp, dsp, bsp, rsp, rsp,
                          _full_spec(h, out_d), _full_spec(1, out_d)],
                out_specs=_row_spec(out_d),
                out_shape=jax.ShapeDtypeStruct((n, out_d), jnp.float32),
            )(agg[0], agg[1], hp, degA, degB, conv_b[i].reshape(1, h), xg0, xt,
              fc_W, fc_b.reshape(1, out_d))
    return out


# speedup vs baseline: 26.2623x; 1.1516x over previous
import functools

import jax
import jax.numpy as jnp
from jax import lax
from jax.experimental import pallas as pl
from jax.experimental.pallas import tpu as pltpu
from jax.experimental.pallas import tpu_sc as plsc

_BN = float(1.0 / (1.0 + 1e-5) ** 0.5)
_RB = 1000
_CH = 128
_NP = 10240
_ZR = 64


def _ln(h, g, b):
    m = jnp.mean(h, axis=-1, keepdims=True)
    v = jnp.mean((h - m) ** 2, axis=-1, keepdims=True)
    return (h - m) * lax.rsqrt(v + 1e-5) * g + b


def _qknorm(y):
    y = jnp.where(y == 0.0, 1e-6, y)
    return y * lax.rsqrt(jnp.sum(y * y, axis=-1, keepdims=True))


def _k1_body(x, tW, tb, g0, b0, gW, gb, cW0, dA, dB, xt0, xg0, hp0):
    xb = x[...]
    h = jnp.dot(xb, tW[...], preferred_element_type=jnp.float32) + tb[...]
    xt0[...] = jnp.maximum(_ln(h, g0[...], b0[...]), 0.0)
    g = jnp.maximum(_BN * (jnp.dot(xb, gW[...], preferred_element_type=jnp.float32) + gb[...]), 0.0)
    xg0[...] = g
    dinv = lax.rsqrt(dA[...] + dB[...] + 1.0)
    hp0[...] = dinv * jnp.dot(g, cW0[...], preferred_element_type=jnp.float32)


def _attn_stats_body(xt, kW, vW, kvs, ksum):
    i = pl.program_id(0)
    xb = xt[...]
    ks = _qknorm(jnp.dot(xb, kW[...], preferred_element_type=jnp.float32))
    vs = jnp.dot(xb, vW[...], preferred_element_type=jnp.float32)
    pkvs = lax.dot_general(ks, vs, (((0,), (0,)), ((), ())),
                           preferred_element_type=jnp.float32)
    pksum = jnp.sum(ks, axis=0, keepdims=True)

    @pl.when(i == 0)
    def _():
        kvs[...] = pkvs
        ksum[...] = pksum

    @pl.when(i != 0)
    def _():
        kvs[...] = kvs[...] + pkvs
        ksum[...] = ksum[...] + pksum


def _attn_apply_body(nn, xt, qW, vW, kvs, ksum, g, b, out):
    xb = xt[...]
    qs = _qknorm(jnp.dot(xb, qW[...], preferred_element_type=jnp.float32))
    vs = jnp.dot(xb, vW[...], preferred_element_type=jnp.float32)
    num = jnp.dot(qs, kvs[...], preferred_element_type=jnp.float32) + nn * vs
    den = jnp.sum(qs * ksum[...], axis=-1, keepdims=True) + nn
    a = (num / den + xb) * 0.5
    out[...] = jnp.maximum(_ln(a, g[...], b[...]), 0.0)


def _gcn_mid_body(aggA, aggB, hp, dA, dB, cb, xg0, cWn, hp_out):
    dinv = lax.rsqrt(dA[...] + dB[...] + 1.0)
    s = dinv * (aggA[...] + aggB[...] + hp[...]) + cb[...]
    xg = jnp.maximum(_BN * s, 0.0) + xg0[...]
    hp_out[...] = dinv * jnp.dot(xg, cWn[...], preferred_element_type=jnp.float32)


def _gcn_final_body(aggA, aggB, hp, dA, dB, cb, xg0, xtf, fW, fb, out):
    dinv = lax.rsqrt(dA[...] + dB[...] + 1.0)
    s = dinv * (aggA[...] + aggB[...] + hp[...]) + cb[...]
    xg = jnp.maximum(_BN * s, 0.0) + xg0[...]
    comb = 0.5 * xg + 0.5 * xtf[...]
    out[...] = jnp.dot(comb, fW[...], preferred_element_type=jnp.float32) + fb[...]


def _row_spec(h):
    return pl.BlockSpec((_RB, h), lambda i: (i, 0))


def _full_spec(r, c):
    return pl.BlockSpec((r, c), lambda i: (0, 0))


def _sc_deg(dst):
    e = dst.shape[0]
    per_w = e // 32
    nfull, tail = divmod(per_w, _CH)

    @functools.partial(
        pl.kernel,
        out_type=jax.ShapeDtypeStruct((2, _NP), jnp.float32),
        mesh=plsc.VectorSubcoreMesh(core_axis_name="c", subcore_axis_name="s"),
        scratch_types=[
            pltpu.VMEM_SHARED((_NP,), jnp.float32),
            pltpu.VMEM((3, _CH), jnp.int32),
            pltpu.VMEM((tail,), jnp.int32),
            pltpu.VMEM((_CH,), jnp.float32),
            pltpu.VMEM((640,), jnp.float32),
            [pltpu.SemaphoreType.DMA] * 3,
            [pltpu.SemaphoreType.DMA] * 2,
        ],
    )
    def k(dst_hbm, out_hbm, acc, idxv, idxt, onesv, zv, isems, ssem):
        c = lax.axis_index("c")
        s = lax.axis_index("s")
        ones16 = jnp.ones((16,), jnp.float32)
        zero16 = jnp.zeros((16,), jnp.float32)
        for r in range(_CH // 16):
            onesv[pl.ds(r * 16, 16)] = ones16
        for r in range(640 // 16):
            zv[pl.ds(r * 16, 16)] = zero16
        pltpu.sync_copy(zv, acc.at[pl.ds(s * 640, 640)])
        plsc.subcore_barrier()
        base0 = (c * 16 + s) * per_w

        def issue_idx(j, ib):
            pltpu.async_copy(dst_hbm.at[pl.ds(base0 + j * _CH, _CH)], idxv.at[ib], isems[ib])

        def wait_idx(j, ib):
            pltpu.make_async_copy(dst_hbm.at[pl.ds(base0 + j * _CH, _CH)], idxv.at[ib], isems[ib]).wait()

        issue_idx(0, 0)

        def six(i, carry):
            for u in range(6):
                j = 6 * i + u
                ib = u % 3
                b = u % 2

                @pl.when(j >= 2)
                def _():
                    pltpu.make_async_copy(onesv, acc.at[idxv.at[(u + 1) % 3]], ssem[b]).wait()

                @pl.when(j + 1 < nfull)
                def _():
                    issue_idx(j + 1, (u + 1) % 3)

                wait_idx(j, ib)
                pltpu.async_copy(onesv, acc.at[idxv.at[ib]], ssem[b], add=True)
            return carry

        lax.fori_loop(0, nfull // 6, six, 0)
        for b in range(2):
            pltpu.make_async_copy(onesv, acc.at[idxv.at[b]], ssem[b]).wait()
        if tail:
            pltpu.sync_copy(dst_hbm.at[pl.ds(base0 + nfull * _CH, tail)], idxt)
            pltpu.sync_copy(onesv.at[pl.ds(0, tail)], acc.at[idxt], add=True)
        plsc.subcore_barrier()
        pltpu.sync_copy(acc.at[pl.ds(s * 640, 640)], out_hbm.at[c, pl.ds(s * 640, 640)])

    return k(dst)


def _sc_agg(h, src, dst, zeros):
    n, hd = h.shape
    e = src.shape[0]
    per_w = e // 32
    nfull, tail = divmod(per_w, _CH)

    assert nfull % 6 == 0, nfull

    @functools.partial(
        pl.kernel,
        out_type=jax.ShapeDtypeStruct((2, _NP, hd), jnp.float32),
        mesh=plsc.VectorSubcoreMesh(core_axis_name="c", subcore_axis_name="s"),
        scratch_types=[
            pltpu.VMEM_SHARED((_NP, hd), jnp.float32),
            pltpu.VMEM((3, _CH), jnp.int32),
            pltpu.VMEM((3, _CH), jnp.int32),
            pltpu.VMEM((_CH, hd), jnp.float32),
            pltpu.VMEM((_CH, hd), jnp.float32),
            pltpu.VMEM((tail,), jnp.int32),
            pltpu.VMEM((tail,), jnp.int32),
            pltpu.VMEM((tail, hd), jnp.float32),
            pltpu.VMEM((_ZR, hd), jnp.float32),
            [pltpu.SemaphoreType.DMA] * 3,
            [pltpu.SemaphoreType.DMA] * 3,
            [pltpu.SemaphoreType.DMA] * 2,
            [pltpu.SemaphoreType.DMA] * 2,
        ],
    )
    def k(h_hbm, src_hbm, dst_hbm, zeros_hbm, out_hbm,
          acc, srci, dsti, rows0, rows1, srct, dstt, rowst, zv,
          isems, jsems, gsem, ssem):
        c = lax.axis_index("c")
        s = lax.axis_index("s")
        rowsv = (rows0, rows1)
        pltpu.sync_copy(zeros_hbm, zv)
        for j in range(10):
            pltpu.sync_copy(zv, acc.at[pl.ds(s * 640 + j * _ZR, _ZR)])
        plsc.subcore_barrier()
        base0 = (c * 16 + s) * per_w

        def issue_idx(j, ib):
            base = base0 + j * _CH
            pltpu.async_copy(src_hbm.at[pl.ds(base, _CH)], srci.at[ib], isems[ib])
            pltpu.async_copy(dst_hbm.at[pl.ds(base, _CH)], dsti.at[ib], jsems[ib])

        def wait_idx(j, ib):
            base = base0 + j * _CH
            pltpu.make_async_copy(src_hbm.at[pl.ds(base, _CH)], srci.at[ib], isems[ib]).wait()
            pltpu.make_async_copy(dst_hbm.at[pl.ds(base, _CH)], dsti.at[ib], jsems[ib]).wait()

        issue_idx(0, 0)
        issue_idx(1, 1)
        pltpu.make_async_copy(src_hbm.at[pl.ds(base0, _CH)], srci.at[0], isems[0]).wait()
        pltpu.make_async_copy(dst_hbm.at[pl.ds(base0, _CH)], dsti.at[0], jsems[0]).wait()
        pltpu.async_copy(h_hbm.at[srci.at[0]], rows0, gsem[0])

        def six(i, carry):
            for u in range(6):
                j = 6 * i + u
                b = u % 2
                ib = u % 3

                @pl.when(j >= 1)
                def _():
                    pltpu.make_async_copy(rowsv[1 - b], acc.at[dsti.at[(u + 2) % 3]], ssem[1 - b]).wait()

                @pl.when(j + 2 < nfull)
                def _():
                    issue_idx(j + 2, (u + 2) % 3)

                @pl.when(j + 1 < nfull)
                def _():
                    wait_idx(j + 1, (u + 1) % 3)
                    pltpu.async_copy(h_hbm.at[srci.at[(u + 1) % 3]], rowsv[1 - b], gsem[1 - b])

                pltpu.make_async_copy(h_hbm.at[srci.at[ib]], rowsv[b], gsem[b]).wait()
                pltpu.async_copy(rowsv[b], acc.at[dsti.at[ib]], ssem[b], add=True)
            return carry

        lax.fori_loop(0, nfull // 6, six, 0)
        pltpu.make_async_copy(rowsv[(nfull - 1) % 2], acc.at[dsti.at[0]], ssem[(nfull - 1) % 2]).wait()
        if tail:
            base = base0 + nfull * _CH
            pltpu.sync_copy(src_hbm.at[pl.ds(base, tail)], srct)
            pltpu.sync_copy(dst_hbm.at[pl.ds(base, tail)], dstt)
            pltpu.async_copy(h_hbm.at[srct], rowst, gsem[0]).wait()
            pltpu.sync_copy(rowst, acc.at[dstt], add=True)
        plsc.subcore_barrier()
        pltpu.sync_copy(acc.at[pl.ds(s * 640, 640)], out_hbm.at[c, pl.ds(s * 640, 640)])

    return k(h, src, dst, zeros)


def kernel(x, edge_index, batch, t_fc_W, t_fc_b, ln_g, ln_b, qW, kW, vW,
           g_fc_W, g_fc_b, conv_W, conv_b, fc_W, fc_b):
    n, d = x.shape
    h = t_fc_W.shape[1]
    out_d = fc_W.shape[1]
    tl = qW.shape[0]
    gl = conv_W.shape[0]
    grid = (n // _RB,)
    del batch

    src = edge_index[0]
    dst = edge_index[1]
    zeros = jnp.zeros((_ZR, h), jnp.float32)

    degp = _sc_deg(dst)
    degA = degp[0].reshape(_NP, 1)
    degB = degp[1].reshape(_NP, 1)

    rsp = _row_spec(h)
    wsp = _full_spec(h, h)
    bsp = _full_spec(1, h)
    dsp = pl.BlockSpec((_RB, 1), lambda i: (i, 0))
    fsd = jax.ShapeDtypeStruct((n, h), jnp.float32)

    xt, xg0, hp = pl.pallas_call(
        _k1_body,
        grid=grid,
        in_specs=[rsp if d == h else _row_spec(d),
                  _full_spec(d, h), bsp, bsp, bsp,
                  _full_spec(d, h), bsp, wsp, dsp, dsp],
        out_specs=[rsp, rsp, rsp],
        out_shape=[fsd, fsd, fsd],
    )(x, t_fc_W, t_fc_b.reshape(1, h), ln_g[0].reshape(1, h), ln_b[0].reshape(1, h),
      g_fc_W, g_fc_b.reshape(1, h), conv_W[0], degA, degB)

    for i in range(tl):
        kvs, ksum = pl.pallas_call(
            _attn_stats_body,
            grid=grid,
            in_specs=[rsp, wsp, wsp],
            out_specs=[_full_spec(h, h), bsp],
            out_shape=[jax.ShapeDtypeStruct((h, h), jnp.float32),
                       jax.ShapeDtypeStruct((1, h), jnp.float32)],
        )(xt, kW[i], vW[i])
        xt = pl.pallas_call(
            functools.partial(_attn_apply_body, float(n)),
            grid=grid,
            in_specs=[rsp, wsp, wsp, _full_spec(h, h), bsp, bsp, bsp],
            out_specs=rsp,
            out_shape=fsd,
        )(xt, qW[i], vW[i], kvs, ksum, ln_g[i + 1].reshape(1, h), ln_b[i + 1].reshape(1, h))

    for i in range(gl):
        agg = _sc_agg(hp, src, dst, zeros)
        if i + 1 < gl:
            hp = pl.pallas_call(
                _gcn_mid_body,
                grid=grid,
                in_specs=[rsp, rsp, rsp, dsp, dsp, bsp, rsp, wsp],
                out_specs=rsp,
                out_shape=fsd,
            )(agg[0], agg[1], hp, degA, degB, conv_b[i].reshape(1, h), xg0, conv_W[i + 1])
        else:
            out = pl.pallas_call(
                _gcn_final_body,
                grid=grid,
                in_specs=[rsp, rsp, rsp, dsp, dsp, bsp, rsp, rsp,
                          _full_spec(h, out_d), _full_spec(1, out_d)],
                out_specs=_row_spec(out_d),
                out_shape=jax.ShapeDtypeStruct((n, out_d), jnp.float32),
            )(agg[0], agg[1], hp, degA, degB, conv_b[i].reshape(1, h), xg0, xt,
              fc_W, fc_b.reshape(1, out_d))
    return out
